# baseline (speedup 1.0000x reference)
import functools

import jax
import jax.numpy as jnp
from jax import lax
from jax.experimental import pallas as pl
from jax.experimental.pallas import tpu as pltpu
from jax.experimental.pallas import tpu_sc as plsc

CS = 16
LANES = 16


@functools.lru_cache(maxsize=None)
def _build(B, S, D, V):
    mesh = plsc.VectorSubcoreMesh(core_axis_name="c", subcore_axis_name="s")
    NC, NS = mesh.num_cores, mesh.num_subcores
    NW = NC * NS
    assert S % (NW * CS) == 0 and D % LANES == 0
    S_PER_W = S // NW
    NCHUNK = S_PER_W // CS
    assert NCHUNK >= 2 and NCHUNK % 2 == 0 and B == 4

    @functools.partial(
        pl.kernel,
        out_type=jax.ShapeDtypeStruct((B * S, D), jnp.float32),
        mesh=mesh,
        scratch_types=[
            pltpu.VMEM((B, S_PER_W), jnp.int32),
            pltpu.VMEM((CS, D), jnp.float32),
            pltpu.VMEM((CS, D), jnp.float32),
            pltpu.VMEM((CS, D), jnp.float32),
            pltpu.VMEM((CS, D), jnp.float32),
            pltpu.VMEM((CS, D), jnp.float32),
            pltpu.VMEM((CS, D), jnp.float32),
        ] + [pltpu.SemaphoreType.DMA] * 10,
    )
    def emb(ids_hbm, tok_hbm, pos_hbm, out_hbm, idx_v, r0_v, r1_v, r2_v, r3_v,
            p0_v, p1_v, g0, g1, g2, g3, o0, o1, o2, o3, ps0, ps1):
        rows = (r0_v, r1_v, r2_v, r3_v)
        pos = (p0_v, p1_v)
        gsem = (g0, g1, g2, g3)
        osem = (o0, o1, o2, o3)
        psem = (ps0, ps1)

        wid = lax.axis_index("s") * NC + lax.axis_index("c")
        s_base = wid * S_PER_W
        smax = S - CS
        omax = S_PER_W - CS

        def start_gather(i_c, b, tb):
            off = jnp.minimum(i_c * CS, omax)
            pltpu.async_copy(
                tok_hbm.at[idx_v.at[b, pl.ds(off, CS)]], rows[tb], gsem[tb])

        def start_pos(i_c, h):
            s0 = jnp.minimum(s_base + i_c * CS, smax)
            pltpu.async_copy(pos_hbm.at[pl.ds(s0, CS)], pos[h], psem[h])

        def add_and_out(i_c, b, h):
            rb = rows[b]
            ph = pos[h]

            UNROLL = 8

            def add_body(r, _):
                def j_body(j, _):
                    for jj in range(UNROLL):
                        c0 = (j * UNROLL + jj) * LANES
                        plsc.addupdate(rb.at[r, pl.ds(c0, LANES)],
                                       ph[r, pl.ds(c0, LANES)])
                    return 0

                lax.fori_loop(0, D // (LANES * UNROLL), j_body, 0)
                return 0

            lax.fori_loop(0, CS, add_body, 0)
            r0 = b * S + s_base + i_c * CS
            pltpu.async_copy(rb, out_hbm.at[pl.ds(r0, CS)], osem[b])

        def chunk(i_c, h, first):
            start_pos(i_c + 1, 1 - h)
            pltpu.make_async_copy(pos_hbm.at[pl.ds(0, CS)], pos[h], psem[h]).wait()
            for b in range(B):
                tb = (b + 2) % 4
                nxt_i = i_c if b < 2 else i_c + 1
                nxt_b = (b + 2) % B
                if not (first and b < 2):
                    pltpu.make_async_copy(
                        rows[tb], out_hbm.at[pl.ds(0, CS)], osem[tb]).wait()
                start_gather(nxt_i, nxt_b, tb)
                pltpu.make_async_copy(
                    tok_hbm.at[idx_v.at[b, pl.ds(0, CS)]], rows[b],
                    gsem[b]).wait()
                add_and_out(i_c, b, h)

        for b in range(B):
            pltpu.sync_copy(ids_hbm.at[pl.ds(b * S + s_base, S_PER_W)],
                            idx_v.at[b])
        start_pos(0, 0)
        start_gather(0, 0, 0)
        start_gather(0, 1, 1)

        chunk(jnp.int32(0), 0, True)
        chunk(jnp.int32(1), 1, False)

        def pair_body(i2, _):
            i_c = 2 + 2 * i2
            chunk(i_c, 0, False)
            chunk(i_c + 1, 1, False)
            return 0

        lax.fori_loop(0, (NCHUNK - 2) // 2, pair_body, 0)

        pltpu.make_async_copy(
            tok_hbm.at[idx_v.at[0, pl.ds(0, CS)]], rows[0], gsem[0]).wait()
        pltpu.make_async_copy(
            tok_hbm.at[idx_v.at[1, pl.ds(0, CS)]], rows[1], gsem[1]).wait()
        pltpu.make_async_copy(rows[2], out_hbm.at[pl.ds(0, CS)], osem[2]).wait()
        pltpu.make_async_copy(rows[3], out_hbm.at[pl.ds(0, CS)], osem[3]).wait()
        pltpu.make_async_copy(pos_hbm.at[pl.ds(0, CS)], pos[0], psem[0]).wait()

    return emb


def kernel(input_ids, token_table, position_table):
    B, S = input_ids.shape
    V, D = token_table.shape
    ids_flat = input_ids.reshape(-1).astype(jnp.int32)
    emb = _build(B, S, D, V)
    out = emb(ids_flat, token_table, position_table)
    return out.reshape(B, S, D)


# speedup vs baseline: 1.1396x; 1.1396x over previous
import functools

import jax
import jax.numpy as jnp
from jax import lax
from jax.experimental import pallas as pl
from jax.experimental.pallas import tpu as pltpu
from jax.experimental.pallas import tpu_sc as plsc

CS = 16
LANES = 16


@functools.lru_cache(maxsize=None)
def _build(B, S, D, V):
    mesh = plsc.VectorSubcoreMesh(core_axis_name="c", subcore_axis_name="s")
    NC, NS = mesh.num_cores, mesh.num_subcores
    NW = NC * NS
    assert S % (NW * CS) == 0 and D % LANES == 0
    S_PER_W = S // NW
    NCHUNK = S_PER_W // CS
    assert NCHUNK >= 2 and NCHUNK % 2 == 0 and B == 4

    @functools.partial(
        pl.kernel,
        out_type=jax.ShapeDtypeStruct((B * S, D), jnp.float32),
        mesh=mesh,
        scratch_types=[
            pltpu.VMEM((B, S_PER_W), jnp.int32),
            pltpu.VMEM((CS, D), jnp.float32),
            pltpu.VMEM((CS, D), jnp.float32),
            pltpu.VMEM((CS, D), jnp.float32),
            pltpu.VMEM((CS, D), jnp.float32),
            pltpu.VMEM((CS, D), jnp.float32),
            pltpu.VMEM((CS, D), jnp.float32),
        ] + [pltpu.SemaphoreType.DMA] * 10,
    )
    def emb(ids_hbm, tok_hbm, pos_hbm, out_hbm, idx_v, r0_v, r1_v, r2_v, r3_v,
            p0_v, p1_v, g0, g1, g2, g3, o0, o1, o2, o3, ps0, ps1):
        rows = (r0_v, r1_v, r2_v, r3_v)
        pos = (p0_v, p1_v)
        gsem = (g0, g1, g2, g3)
        osem = (o0, o1, o2, o3)
        psem = (ps0, ps1)

        wid = lax.axis_index("s") * NC + lax.axis_index("c")
        s_base = wid * S_PER_W
        smax = S - CS
        omax = S_PER_W - CS

        def start_gather(i_c, b, tb):
            off = jnp.minimum(i_c * CS, omax)
            pltpu.async_copy(
                tok_hbm.at[idx_v.at[b, pl.ds(off, CS)]], rows[tb], gsem[tb])

        def start_pos(i_c, h):
            s0 = jnp.minimum(s_base + i_c * CS, smax)
            pltpu.async_copy(pos_hbm.at[pl.ds(s0, CS)], pos[h], psem[h])

        def add_and_out(i_c, b, h):
            rb = rows[b]
            ph = pos[h]

            def add_body(r, _):
                for j in range(D // LANES):
                    plsc.addupdate(rb.at[r, pl.ds(j * LANES, LANES)],
                                   ph[r, pl.ds(j * LANES, LANES)])
                return 0

            lax.fori_loop(0, CS, add_body, 0)
            r0 = b * S + s_base + i_c * CS
            pltpu.async_copy(rb, out_hbm.at[pl.ds(r0, CS)], osem[b])

        def chunk(i_c, h, guard):
            start_pos(i_c + 1, 1 - h)
            pltpu.make_async_copy(pos_hbm.at[pl.ds(0, CS)], pos[h], psem[h]).wait()
            for b in range(B):
                tb = (b + 2) % 4
                nxt_i = i_c if b < 2 else i_c + 1
                nxt_b = (b + 2) % B

                def _drain_out():
                    pltpu.make_async_copy(
                        rows[tb], out_hbm.at[pl.ds(0, CS)], osem[tb]).wait()

                if guard is not None and b < 2:
                    pl.when(guard)(_drain_out)
                else:
                    _drain_out()
                start_gather(nxt_i, nxt_b, tb)
                pltpu.make_async_copy(
                    tok_hbm.at[idx_v.at[b, pl.ds(0, CS)]], rows[b],
                    gsem[b]).wait()
                add_and_out(i_c, b, h)

        for b in range(B):
            pltpu.sync_copy(ids_hbm.at[pl.ds(b * S + s_base, S_PER_W)],
                            idx_v.at[b])
        start_pos(0, 0)
        start_gather(0, 0, 0)
        start_gather(0, 1, 1)

        def pair_body(i2, _):
            i_c = 2 * i2
            chunk(i_c, 0, i2 > 0)
            chunk(i_c + 1, 1, None)
            return 0

        lax.fori_loop(0, NCHUNK // 2, pair_body, 0)

        pltpu.make_async_copy(
            tok_hbm.at[idx_v.at[0, pl.ds(0, CS)]], rows[0], gsem[0]).wait()
        pltpu.make_async_copy(
            tok_hbm.at[idx_v.at[1, pl.ds(0, CS)]], rows[1], gsem[1]).wait()
        pltpu.make_async_copy(rows[2], out_hbm.at[pl.ds(0, CS)], osem[2]).wait()
        pltpu.make_async_copy(rows[3], out_hbm.at[pl.ds(0, CS)], osem[3]).wait()
        pltpu.make_async_copy(pos_hbm.at[pl.ds(0, CS)], pos[0], psem[0]).wait()

    return emb


def kernel(input_ids, token_table, position_table):
    B, S = input_ids.shape
    V, D = token_table.shape
    ids_flat = input_ids.reshape(-1).astype(jnp.int32)
    emb = _build(B, S, D, V)
    out = emb(ids_flat, token_table, position_table)
    return out.reshape(B, S, D)


# speedup vs baseline: 1.9353x; 1.6982x over previous
import functools

import jax
import jax.numpy as jnp
from jax import lax
from jax.experimental import pallas as pl
from jax.experimental.pallas import tpu as pltpu
from jax.experimental.pallas import tpu_sc as plsc

CS = 16
LANES = 16


@functools.lru_cache(maxsize=None)
def _build(B, S, D, V):
    mesh = plsc.VectorSubcoreMesh(core_axis_name="c", subcore_axis_name="s")
    NC, NS = mesh.num_cores, mesh.num_subcores
    NW = NC * NS
    assert S % (NW * CS) == 0 and D % LANES == 0
    S_PER_W = S // NW
    NCHUNK = S_PER_W // CS
    assert NCHUNK >= 2 and NCHUNK % 2 == 0 and B == 4

    @functools.partial(
        pl.kernel,
        out_type=jax.ShapeDtypeStruct((B * S, D), jnp.float32),
        mesh=mesh,
        scratch_types=[
            pltpu.VMEM((4, CS), jnp.int32),
            pltpu.VMEM((CS, D), jnp.float32),
            pltpu.VMEM((CS, D), jnp.float32),
            pltpu.VMEM((CS, D), jnp.float32),
            pltpu.VMEM((CS, D), jnp.float32),
            pltpu.VMEM((CS, D), jnp.float32),
            pltpu.VMEM((CS, D), jnp.float32),
        ] + [pltpu.SemaphoreType.DMA] * 10,
    )
    def emb(ids_hbm, tok_hbm, pos_hbm, out_hbm, idx_v, r0_v, r1_v, r2_v, r3_v,
            p0_v, p1_v, g0, g1, g2, g3, o0, o1, o2, o3, ps0, ps1):
        rows = (r0_v, r1_v, r2_v, r3_v)
        pos = (p0_v, p1_v)
        gsem = (g0, g1, g2, g3)
        osem = (o0, o1, o2, o3)
        psem = (ps0, ps1)

        wid = lax.axis_index("s") * NC + lax.axis_index("c")
        s_base = wid * S_PER_W
        smax = S - CS
        omax = S_PER_W - CS

        def start_gather(i_c, b, tb):
            s0 = jnp.minimum(s_base + i_c * CS, smax)
            r0 = b * S + s0
            pltpu.sync_copy(ids_hbm.at[pl.ds(r0, CS)], idx_v.at[tb])
            pltpu.async_copy(tok_hbm.at[idx_v.at[tb]], rows[tb], gsem[tb])

        def start_pos(i_c, h):
            s0 = jnp.minimum(s_base + i_c * CS, smax)
            pltpu.async_copy(pos_hbm.at[pl.ds(s0, CS)], pos[h], psem[h])

        def add_and_out(i_c, b, h):
            rb = rows[b]
            ph = pos[h]

            def add_body(r, _):
                for j in range(D // LANES):
                    plsc.addupdate(rb.at[r, pl.ds(j * LANES, LANES)],
                                   ph[r, pl.ds(j * LANES, LANES)])
                return 0

            lax.fori_loop(0, CS, add_body, 0)
            r0 = b * S + s_base + i_c * CS
            pltpu.async_copy(rb, out_hbm.at[pl.ds(r0, CS)], osem[b])

        def chunk(i_c, h, guard):
            start_pos(i_c + 1, 1 - h)
            pltpu.make_async_copy(pos_hbm.at[pl.ds(0, CS)], pos[h], psem[h]).wait()
            for b in range(B):
                tb = (b + 2) % 4
                nxt_i = i_c if b < 2 else i_c + 1
                nxt_b = (b + 2) % B

                def _drain_out():
                    pltpu.make_async_copy(
                        rows[tb], out_hbm.at[pl.ds(0, CS)], osem[tb]).wait()

                if guard is not None and b < 2:
                    pl.when(guard)(_drain_out)
                else:
                    _drain_out()
                start_gather(nxt_i, nxt_b, tb)
                pltpu.make_async_copy(
                    tok_hbm.at[idx_v.at[b]], rows[b], gsem[b]).wait()
                add_and_out(i_c, b, h)

        start_pos(0, 0)
        start_gather(0, 0, 0)
        start_gather(0, 1, 1)

        def pair_body(i2, _):
            i_c = 2 * i2
            chunk(i_c, 0, i2 > 0)
            chunk(i_c + 1, 1, None)
            return 0

        lax.fori_loop(0, NCHUNK // 2, pair_body, 0)

        pltpu.make_async_copy(tok_hbm.at[idx_v.at[0]], rows[0], gsem[0]).wait()
        pltpu.make_async_copy(tok_hbm.at[idx_v.at[1]], rows[1], gsem[1]).wait()
        pltpu.make_async_copy(rows[2], out_hbm.at[pl.ds(0, CS)], osem[2]).wait()
        pltpu.make_async_copy(rows[3], out_hbm.at[pl.ds(0, CS)], osem[3]).wait()
        pltpu.make_async_copy(pos_hbm.at[pl.ds(0, CS)], pos[0], psem[0]).wait()

    return emb


def kernel(input_ids, token_table, position_table):
    B, S = input_ids.shape
    V, D = token_table.shape
    ids_flat = input_ids.reshape(-1).astype(jnp.int32)
    emb = _build(B, S, D, V)
    out = emb(ids_flat, token_table, position_table)
    return out.reshape(B, S, D)
